# Initial kernel scaffold; baseline (speedup 1.0000x reference)
import jax
import jax.numpy as jnp
from jax.experimental import pallas as pl


def kernel(x, edge_index, edge_weight, emb, W0, b0, W1, b1, W2, b2, gw0, gb0, gm0, gw1, gb1, gm1):
    raise NotImplementedError("write your pallas kernel here")


# speedup vs baseline: 1.2804x; 1.2804x over previous
import functools

import jax
import jax.numpy as jnp
from jax.experimental import pallas as pl
from jax.experimental.pallas import tpu as pltpu

N_NODES = 10000
HIDDEN = 128
EPS = 1e-5


def _post_norm_body(agg_ref, hw_ref, dinv2_ref, b_ref, gw_ref, gb_ref, gm_ref,
                    hn_ref, hr_ref):
    t = agg_ref[...] + dinv2_ref[...] * hw_ref[...] + b_ref[...]
    mean = jnp.mean(t, axis=0, keepdims=True)
    c = t - mean * gm_ref[...]
    var = jnp.mean(c * c, axis=0, keepdims=True)
    hn = gw_ref[...] * c * jax.lax.rsqrt(var + EPS) + gb_ref[...]
    hn_ref[...] = hn
    hr_ref[...] = jnp.maximum(hn, 0.0)


@jax.jit
def _post_norm(agg, hw, dinv2, b, gw, gb, gm):
    return pl.pallas_call(
        _post_norm_body,
        out_shape=(jax.ShapeDtypeStruct((N_NODES, HIDDEN), jnp.float32),
                   jax.ShapeDtypeStruct((N_NODES, HIDDEN), jnp.float32)),
    )(agg, hw, dinv2, b.reshape(1, -1), gw.reshape(1, -1),
      gb.reshape(1, -1), gm.reshape(1, -1))


def _post_last_body(agg_ref, hw_ref, dinv2_ref, b_ref, o_ref):
    o_ref[...] = agg_ref[...] + dinv2_ref[...] * hw_ref[...] + b_ref[...]


@jax.jit
def _post_last(agg, hw, dinv2, b):
    return pl.pallas_call(
        _post_last_body,
        out_shape=jax.ShapeDtypeStruct((N_NODES, HIDDEN), jnp.float32),
    )(agg, hw, dinv2, b.reshape(1, -1))


def _matmul_body(h_ref, w_ref, o_ref):
    o_ref[...] = jnp.dot(h_ref[...], w_ref[...],
                         preferred_element_type=jnp.float32)


@jax.jit
def _matmul(h, w):
    return pl.pallas_call(
        _matmul_body,
        out_shape=jax.ShapeDtypeStruct((h.shape[0], w.shape[1]), jnp.float32),
    )(h, w)


def kernel(x, edge_index, edge_weight, emb, W0, b0, W1, b1, W2, b2,
           gw0, gb0, gm0, gw1, gb1, gm1):
    row = edge_index[0]
    col = edge_index[1]
    n = N_NODES

    deg = jnp.ones((n,), jnp.float32).at[col].add(edge_weight)
    dinv = jax.lax.rsqrt(deg)
    norm = dinv[row] * edge_weight * dinv[col]
    dinv2 = (dinv * dinv).reshape(n, 1)

    table0 = _matmul(emb, W0)
    hw0 = table0[x]

    def agg(hw):
        msg = hw[row] * norm[:, None]
        return jnp.zeros((n, HIDDEN), jnp.float32).at[col].add(msg)

    h1, h1r = _post_norm(agg(hw0), hw0, dinv2, b0, gw0, gb0, gm0)
    hw1 = _matmul(h1r, W1)
    h2, h2r = _post_norm(agg(hw1), hw1, dinv2, b1, gw1, gb1, gm1)
    hw2 = _matmul(h2r, W2)
    h3 = _post_last(agg(hw2), hw2, dinv2, b2)
    return jnp.concatenate([h1, h2, h3], axis=-1)


# speedup vs baseline: 7.2497x; 5.6622x over previous
import functools

import jax
import jax.numpy as jnp
from jax import lax
from jax.experimental import pallas as pl
from jax.experimental.pallas import tpu as pltpu
from jax.experimental.pallas import tpu_sc as plsc

N = 10000
NP = 10240
H = 128
E = 320000
NC = 2
NS = 16
NW = NC * NS
L = 16
CHUNK = 128
NCHUNK = 79
EPW = NCHUNK * CHUNK
E_PAD = EPW * NW
RPT = NP // NS
EPS = 1e-5

_mesh = plsc.VectorSubcoreMesh(core_axis_name="c", subcore_axis_name="s")


@functools.partial(
    pl.kernel,
    out_type=jax.ShapeDtypeStruct((NC, NP, L), jnp.float32),
    mesh=_mesh,
    scratch_types=[
        pltpu.VMEM((NCHUNK, CHUNK), jnp.int32),
        pltpu.VMEM((EPW,), jnp.float32),
        pltpu.VMEM((CHUNK, L), jnp.float32),
        pltpu.VMEM_SHARED((NP, L), jnp.float32),
    ],
    compiler_params=pltpu.CompilerParams(needs_layout_passes=False),
)
def _deg_kernel(col_hbm, w_hbm, out_hbm, col_v, wf_v, msg_v, acc_sh):
    cid = lax.axis_index("c")
    sid = lax.axis_index("s")
    wid = cid * NS + sid

    pltpu.sync_copy(col_hbm.at[wid], col_v)
    pltpu.sync_copy(w_hbm.at[wid], wf_v)

    def zrow(i, _):
        msg_v[i, :] = jnp.zeros((L,), jnp.float32)
        return 0
    lax.fori_loop(0, CHUNK, zrow, 0)

    base = sid * RPT
    for k in range(5):
        pltpu.sync_copy(msg_v, acc_sh.at[pl.ds(base + k * CHUNK, CHUNK)])
    plsc.subcore_barrier()

    def chunk_body(j, _):
        def erow(e, _):
            wb = plsc.load_gather(wf_v, [jnp.full((L,), j * CHUNK + e,
                                                  jnp.int32)])
            msg_v[e, :] = wb
            return 0
        lax.fori_loop(0, CHUNK, erow, 0)
        pltpu.sync_copy(msg_v, acc_sh.at[col_v.at[j]], add=True)
        return 0
    lax.fori_loop(0, NCHUNK, chunk_body, 0)

    plsc.subcore_barrier()
    pltpu.sync_copy(acc_sh.at[pl.ds(base, RPT)],
                    out_hbm.at[cid, pl.ds(base, RPT)])


@functools.partial(
    pl.kernel,
    out_type=jax.ShapeDtypeStruct((NC, NP, H), jnp.float32),
    mesh=_mesh,
    scratch_types=[
        pltpu.VMEM((3, CHUNK), jnp.int32),
        pltpu.VMEM((NP,), jnp.float32),
        pltpu.VMEM((CHUNK,), jnp.float32),
        pltpu.VMEM((CHUNK, H), jnp.float32),
        pltpu.VMEM_SHARED((NP, H), jnp.float32),
        pltpu.SemaphoreType.DMA,
    ],
    compiler_params=pltpu.CompilerParams(needs_layout_passes=False),
)
def _agg_kernel(hw_hbm, ec_hbm, dinv_hbm, out_hbm,
                ec_v, dinv_v, nrm_v, buf, acc_sh, sem):
    cid = lax.axis_index("c")
    sid = lax.axis_index("s")
    wid = cid * NS + sid

    pltpu.sync_copy(dinv_hbm, dinv_v)

    zeros16 = jnp.zeros((L,), jnp.float32)

    def zrow(i, _):
        for g in range(H // L):
            buf[i, pl.ds(g * L, L)] = zeros16
        return 0
    lax.fori_loop(0, CHUNK, zrow, 0)

    base = sid * RPT
    for k in range(5):
        pltpu.sync_copy(buf, acc_sh.at[pl.ds(base + k * CHUNK, CHUNK)])
    plsc.subcore_barrier()

    def chunk_body(j, _):
        pltpu.sync_copy(ec_hbm.at[wid, j], ec_v)
        pltpu.async_copy(hw_hbm.at[ec_v.at[0]], buf, sem).wait()
        for g in range(CHUNK // L):
            r16 = ec_v[0, pl.ds(g * L, L)]
            c16 = ec_v[1, pl.ds(g * L, L)]
            w16 = plsc.bitcast(ec_v[2, pl.ds(g * L, L)], jnp.float32)
            n16 = (plsc.load_gather(dinv_v, [r16]) * w16 *
                   plsc.load_gather(dinv_v, [c16]))
            nrm_v[pl.ds(g * L, L)] = n16

        def erow(e, _):
            nb = plsc.load_gather(nrm_v, [jnp.full((L,), e, jnp.int32)])
            for g in range(H // L):
                buf[e, pl.ds(g * L, L)] = buf[e, pl.ds(g * L, L)] * nb
            return 0
        lax.fori_loop(0, CHUNK, erow, 0)

        pltpu.sync_copy(buf, acc_sh.at[ec_v.at[1]], add=True)
        return 0
    lax.fori_loop(0, NCHUNK, chunk_body, 0)

    plsc.subcore_barrier()
    pltpu.sync_copy(acc_sh.at[pl.ds(base, RPT)],
                    out_hbm.at[cid, pl.ds(base, RPT)])


def _lift_body(emb_ref, w0_ref, x_ref, hw0_ref):
    t0 = jnp.dot(emb_ref[...], w0_ref[...],
                 preferred_element_type=jnp.float32)
    xoh = (x_ref[...] == lax.broadcasted_iota(jnp.int32, (1, emb_ref.shape[0]), 1)
           ).astype(jnp.float32)
    hw0_ref[...] = jnp.dot(xoh, t0, preferred_element_type=jnp.float32)


@jax.jit
def _lift(emb, w0, x2):
    return pl.pallas_call(
        _lift_body,
        out_shape=jax.ShapeDtypeStruct((N, H), jnp.float32),
    )(emb, w0, x2)


def _dinv_body(p_ref, dinv_ref, dinv2_ref):
    d = p_ref[0, :, 0] + p_ref[1, :, 0] + 1.0
    di = lax.rsqrt(d)
    dinv_ref[...] = di[:, None]
    dinv2_ref[...] = (di * di)[:N, None]


@jax.jit
def _dinv(degp):
    return pl.pallas_call(
        _dinv_body,
        out_shape=(jax.ShapeDtypeStruct((NP, 1), jnp.float32),
                   jax.ShapeDtypeStruct((N, 1), jnp.float32)),
    )(degp)


def _post_body(p_ref, hw_ref, dinv2_ref, b_ref, gw_ref, gb_ref, gm_ref,
               wn_ref, hn_ref, hwn_ref):
    t = (p_ref[0, :N] + p_ref[1, :N]
         + dinv2_ref[...] * hw_ref[...] + b_ref[...])
    mean = jnp.mean(t, axis=0, keepdims=True)
    c = t - mean * gm_ref[...]
    var = jnp.mean(c * c, axis=0, keepdims=True)
    hn = gw_ref[...] * c * lax.rsqrt(var + EPS) + gb_ref[...]
    hn_ref[...] = hn
    hwn_ref[...] = jnp.dot(jnp.maximum(hn, 0.0), wn_ref[...],
                           preferred_element_type=jnp.float32)


@jax.jit
def _post(p, hw, dinv2, b, gw, gb, gm, wn):
    return pl.pallas_call(
        _post_body,
        out_shape=(jax.ShapeDtypeStruct((N, H), jnp.float32),
                   jax.ShapeDtypeStruct((N, H), jnp.float32)),
    )(p, hw, dinv2, b.reshape(1, -1), gw.reshape(1, -1),
      gb.reshape(1, -1), gm.reshape(1, -1), wn)


def _post_last_body(p_ref, hw_ref, dinv2_ref, b_ref, o_ref):
    o_ref[...] = (p_ref[0, :N] + p_ref[1, :N]
                  + dinv2_ref[...] * hw_ref[...] + b_ref[...])


@jax.jit
def _post_last(p, hw, dinv2, b):
    return pl.pallas_call(
        _post_last_body,
        out_shape=jax.ShapeDtypeStruct((N, H), jnp.float32),
    )(p, hw, dinv2, b.reshape(1, -1))


def kernel(x, edge_index, edge_weight, emb, W0, b0, W1, b1, W2, b2,
           gw0, gb0, gm0, gw1, gb1, gm1):
    pad = E_PAD - E
    row_t = jnp.pad(edge_index[0].astype(jnp.int32),
                    (0, pad)).reshape(NW, NCHUNK, CHUNK)
    col_t = jnp.pad(edge_index[1].astype(jnp.int32),
                    (0, pad)).reshape(NW, NCHUNK, CHUNK)
    w_t = jnp.pad(edge_weight, (0, pad)).reshape(NW, NCHUNK, CHUNK)
    x2 = x.astype(jnp.int32).reshape(N, 1)

    w_bits = lax.bitcast_convert_type(w_t, jnp.int32)
    ec = jnp.stack([row_t, col_t, w_bits], axis=2)

    hw0 = _lift(emb, W0, x2)
    degp = _deg_kernel(col_t, w_t.reshape(NW, EPW))
    dinv2d, dinv2 = _dinv(degp)
    dinv = dinv2d.reshape(NP)

    p0 = _agg_kernel(hw0, ec, dinv)
    h1, hw1 = _post(p0, hw0, dinv2, b0, gw0, gb0, gm0, W1)
    p1 = _agg_kernel(hw1, ec, dinv)
    h2, hw2 = _post(p1, hw1, dinv2, b1, gw1, gb1, gm1, W2)
    p2 = _agg_kernel(hw2, ec, dinv)
    h3 = _post_last(p2, hw2, dinv2, b2)
    return jnp.concatenate([h1, h2, h3], axis=-1)
